# baseline (speedup 1.0000x reference)
import functools

import jax
import jax.numpy as jnp
from jax import lax
from jax.experimental import pallas as pl
from jax.experimental.pallas import tpu as pltpu
from jax.experimental.pallas import tpu_sc as plsc

B = 4096
L = 200
D = 64
T = B * L

NW = 32
PER_W = T // NW
C = 128
NSLOT = 4
MACRO = PER_W // (C * NSLOT)

_mesh = plsc.VectorSubcoreMesh(core_axis_name="c", subcore_axis_name="s")


@functools.partial(
    pl.kernel,
    mesh=_mesh,
    compiler_params=pltpu.CompilerParams(
        use_tc_tiling_on_sc=False, needs_layout_passes=False
    ),
    out_type=jax.ShapeDtypeStruct((T, D), jnp.float32),
    scratch_types=[
        pltpu.VMEM((NSLOT, C), jnp.int32),
        pltpu.VMEM((NSLOT, C), jnp.int32),
        pltpu.VMEM((NSLOT, C), jnp.int32),
        pltpu.VMEM((NSLOT, C, D), jnp.float32),
        pltpu.VMEM((512, D), jnp.float32),
        pltpu.VMEM((512, D), jnp.float32),
    ]
    + [pltpu.SemaphoreType.DMA] * (3 * NSLOT),
)
def _emb_lookup(tok_hbm, row_hbm, col_hbm, ttab, rtab, ctab, out_hbm,
                idx_t, idx_r, idx_c, buf_t, rtab_l, ctab_l, *sems):
    s_idx = sems[0:NSLOT]
    s_gat = sems[NSLOT:2 * NSLOT]
    s_out = sems[2 * NSLOT:3 * NSLOT]
    wid = lax.axis_index("s") * 2 + lax.axis_index("c")
    base0 = wid * PER_W

    def issue_idx(j, chunk):
        src = pl.ds(base0 + chunk * C, C)
        pltpu.async_copy(tok_hbm.at[src], idx_t.at[j], s_idx[j])
        pltpu.async_copy(row_hbm.at[src], idx_r.at[j], s_idx[j])
        pltpu.async_copy(col_hbm.at[src], idx_c.at[j], s_idx[j])

    def wait_idx(j):
        pltpu.make_async_copy(tok_hbm.at[pl.ds(0, C)], idx_t.at[j], s_idx[j]).wait()
        pltpu.make_async_copy(row_hbm.at[pl.ds(0, C)], idx_r.at[j], s_idx[j]).wait()
        pltpu.make_async_copy(col_hbm.at[pl.ds(0, C)], idx_c.at[j], s_idx[j]).wait()

    def issue_gather(j):
        pltpu.async_copy(ttab.at[idx_t.at[j]], buf_t.at[j], s_gat[j])

    def wait_gather(j):
        pltpu.make_async_copy(ttab.at[idx_t.at[j]], buf_t.at[j], s_gat[j]).wait()

    def issue_out(j, chunk):
        dst = pl.ds(base0 + chunk * C, C)
        pltpu.async_copy(buf_t.at[j], out_hbm.at[dst], s_out[j])

    def wait_out(j):
        pltpu.make_async_copy(buf_t.at[j], out_hbm.at[pl.ds(0, C)], s_out[j]).wait()

    pltpu.sync_copy(rtab, rtab_l)
    pltpu.sync_copy(ctab, ctab_l)

    for j in range(NSLOT):
        issue_idx(j, j)
    for j in range(NSLOT):
        wait_idx(j)
        issue_gather(j)

    iota16 = lax.broadcasted_iota(jnp.int32, (16,), 0)

    def macro_body(m, carry):
        chunk0 = m * NSLOT
        for j in range(NSLOT):
            wait_gather(j)

            def blk_body(ib, c2):
                rows = iota16 + ib * 16
                ir = idx_r[j, pl.ds(ib * 16, 16)]
                ic = idx_c[j, pl.ds(ib * 16, 16)]
                for d in range(D):
                    dvec = jnp.full((16,), d, jnp.int32)
                    vr = plsc.load_gather(rtab_l, [ir, dvec])
                    vc = plsc.load_gather(ctab_l, [ic, dvec])
                    plsc.addupdate_scatter(buf_t.at[j], [rows, dvec], vr + vc)
                return c2

            lax.fori_loop(0, C // 16, blk_body, 0)
            issue_out(j, chunk0 + j)
            @pl.when(m < MACRO - 1)
            def _():
                issue_idx(j, chunk0 + NSLOT + j)

        @pl.when(m < MACRO - 1)
        def _():
            for j in range(NSLOT):
                wait_idx(j)
                wait_out(j)
                issue_gather(j)

        return carry

    lax.fori_loop(0, MACRO, macro_body, 0)
    for j in range(NSLOT):
        wait_out(j)


def kernel(tokens, row_indices, col_indices, token_table, row_table, col_table):
    tok = tokens.reshape(T).astype(jnp.int32)
    ri = row_indices.reshape(T).astype(jnp.int32)
    ci = col_indices.reshape(T).astype(jnp.int32)
    out = _emb_lookup(tok, ri, ci, token_table, row_table, col_table)
    return out.reshape(B, L, D)


# speedup vs baseline: 2.4950x; 2.4950x over previous
import functools

import jax
import jax.numpy as jnp
from jax import lax
from jax.experimental import pallas as pl
from jax.experimental.pallas import tpu as pltpu
from jax.experimental.pallas import tpu_sc as plsc

B = 4096
L = 200
D = 64
T = B * L

NW = 32
PER_W = T // NW
C = 128
NSLOT = 4
MACRO = PER_W // (C * NSLOT)

_mesh = plsc.VectorSubcoreMesh(core_axis_name="c", subcore_axis_name="s")


@functools.partial(
    pl.kernel,
    mesh=_mesh,
    compiler_params=pltpu.CompilerParams(
        use_tc_tiling_on_sc=False, needs_layout_passes=False
    ),
    out_type=jax.ShapeDtypeStruct((T, D), jnp.float32),
    scratch_types=[
        pltpu.VMEM((NSLOT, C), jnp.int32),
        pltpu.VMEM((NSLOT, C), jnp.int32),
        pltpu.VMEM((NSLOT, C), jnp.int32),
        pltpu.VMEM((NSLOT, C, D), jnp.float32),
        pltpu.VMEM((512, D), jnp.float32),
        pltpu.VMEM((512, D), jnp.float32),
    ]
    + [pltpu.SemaphoreType.DMA] * (3 * NSLOT),
)
def _emb_lookup(tok_hbm, row_hbm, col_hbm, ttab, rtab, ctab, out_hbm,
                idx_t, idx_r, idx_c, buf_t, rtab_l, ctab_l, *sems):
    s_idx = sems[0:NSLOT]
    s_gat = sems[NSLOT:2 * NSLOT]
    s_out = sems[2 * NSLOT:3 * NSLOT]
    wid = lax.axis_index("s") * 2 + lax.axis_index("c")
    base0 = wid * PER_W

    def issue_idx(j, chunk):
        src = pl.ds(base0 + chunk * C, C)
        pltpu.async_copy(tok_hbm.at[src], idx_t.at[j], s_idx[j])
        pltpu.async_copy(row_hbm.at[src], idx_r.at[j], s_idx[j])
        pltpu.async_copy(col_hbm.at[src], idx_c.at[j], s_idx[j])

    def wait_idx(j):
        pltpu.make_async_copy(tok_hbm.at[pl.ds(0, C)], idx_t.at[j], s_idx[j]).wait()
        pltpu.make_async_copy(row_hbm.at[pl.ds(0, C)], idx_r.at[j], s_idx[j]).wait()
        pltpu.make_async_copy(col_hbm.at[pl.ds(0, C)], idx_c.at[j], s_idx[j]).wait()

    def issue_gather(j):
        pltpu.async_copy(ttab.at[idx_t.at[j]], buf_t.at[j], s_gat[j])

    def wait_gather(j):
        pltpu.make_async_copy(ttab.at[idx_t.at[j]], buf_t.at[j], s_gat[j]).wait()

    def issue_out(j, chunk):
        dst = pl.ds(base0 + chunk * C, C)
        pltpu.async_copy(buf_t.at[j], out_hbm.at[dst], s_out[j])

    def wait_out(j):
        pltpu.make_async_copy(buf_t.at[j], out_hbm.at[pl.ds(0, C)], s_out[j]).wait()

    pltpu.sync_copy(rtab, rtab_l)
    pltpu.sync_copy(ctab, ctab_l)

    for j in range(NSLOT):
        issue_idx(j, j)
    for j in range(NSLOT):
        wait_idx(j)
        issue_gather(j)

    def macro_body(m, carry):
        chunk0 = m * NSLOT
        for j in range(NSLOT):
            wait_gather(j)

            def blk_body(ib, c2):
                i0 = ib * 16
                ir16 = idx_r[j, pl.ds(i0, 16)]
                ic16 = idx_c[j, pl.ds(i0, 16)]
                for k in range(16):
                    ri = ir16[k]
                    ci = ic16[k]
                    for dd in range(D // 16):
                        sl = pl.ds(dd * 16, 16)
                        v = rtab_l[ri, sl] + ctab_l[ci, sl]
                        plsc.addupdate(buf_t.at[j, i0 + k, sl], v)
                return c2

            lax.fori_loop(0, C // 16, blk_body, 0)
            issue_out(j, chunk0 + j)
            @pl.when(m < MACRO - 1)
            def _():
                issue_idx(j, chunk0 + NSLOT + j)

        @pl.when(m < MACRO - 1)
        def _():
            for j in range(NSLOT):
                wait_idx(j)
                wait_out(j)
                issue_gather(j)

        return carry

    lax.fori_loop(0, MACRO, macro_body, 0)
    for j in range(NSLOT):
        wait_out(j)


def kernel(tokens, row_indices, col_indices, token_table, row_table, col_table):
    tok = tokens.reshape(T).astype(jnp.int32)
    ri = row_indices.reshape(T).astype(jnp.int32)
    ci = col_indices.reshape(T).astype(jnp.int32)
    out = _emb_lookup(tok, ri, ci, token_table, row_table, col_table)
    return out.reshape(B, L, D)
